# baseline (speedup 1.0000x reference)
import functools

import jax
import jax.numpy as jnp
from jax import lax
from jax.experimental import pallas as pl
from jax.experimental.pallas import tpu as pltpu
from jax.experimental.pallas import tpu_sc as plsc

_CHUNK = 128
_NW = 32
_NSUB = 16


def _edge_agg(y, src2d, dst2d, zeros, n_pad, chunk, segs0, segs1):
    n, d = y.shape
    t_chunks = src2d.shape[0]
    n_stage = t_chunks // (_NSUB * (segs0 + segs1))
    rows_per_tile = n_pad // _NSUB
    mesh = plsc.VectorSubcoreMesh(core_axis_name="c", subcore_axis_name="s")

    @functools.partial(
        pl.kernel,
        mesh=mesh,
        compiler_params=pltpu.CompilerParams(use_tc_tiling_on_sc=False),
        out_type=jax.ShapeDtypeStruct((2 * n_pad, d), jnp.float32),
        scratch_types=[
            pltpu.VMEM((n_stage, chunk), jnp.int32),
            pltpu.VMEM((n_stage, chunk), jnp.int32),
            pltpu.VMEM((4, chunk, d), jnp.float32),
            pltpu.VMEM_SHARED((n_pad, d), jnp.float32),
            [pltpu.SemaphoreType.DMA] * 4,
            [pltpu.SemaphoreType.DMA] * 4,
        ],
    )
    def k(y_hbm, src_hbm, dst_hbm, z_hbm, out_hbm, src_v, dst_v, rows_v,
          agg_s, gsems, ssems):
        c = lax.axis_index("c")
        s = lax.axis_index("s")
        with jax.named_scope("zero"):
            pltpu.sync_copy(z_hbm,
                            agg_s.at[pl.ds(s * rows_per_tile, rows_per_tile)])
            plsc.subcore_barrier()

        def gather(j, p):
            pltpu.async_copy(y_hbm.at[src_v.at[j]], rows_v.at[p], gsems[p])

        def wait_gather(j, p):
            pltpu.make_async_copy(y_hbm.at[src_v.at[j]], rows_v.at[p],
                                  gsems[p]).wait()

        def scatter(j, p):
            pltpu.async_copy(rows_v.at[p], agg_s.at[dst_v.at[j]], ssems[p],
                             add=True)

        def wait_scatter(j, p):
            pltpu.make_async_copy(rows_v.at[p], agg_s.at[dst_v.at[j]],
                                  ssems[p]).wait()

        def run_segment(base):
            with jax.named_scope("stage"):
                pltpu.sync_copy(src_hbm.at[pl.ds(base, n_stage)], src_v)
                pltpu.sync_copy(dst_hbm.at[pl.ds(base, n_stage)], dst_v)

            gather(0, 0)
            gather(1, 1)
            for j in (0, 1):
                wait_gather(j, j)
                scatter(j, j)
                gather(j + 2, j + 2)

            def body(jj, carry):
                for u in range(4):
                    j = 4 * jj + 2 + u
                    p = (2 + u) % 4
                    wait_gather(j, p)
                    scatter(j, p)
                    wait_scatter(j - 2, (p + 2) % 4)
                    gather(j + 2, (p + 2) % 4)
                return carry

            with jax.named_scope("pipe"):
                lax.fori_loop(0, (n_stage - 4) // 4, body, 0)

            for j in (n_stage - 2, n_stage - 1):
                p = j % 4
                wait_gather(j, p)
                scatter(j, p)
                wait_scatter(j - 2, (p + 2) % 4)
            for j in (n_stage - 2, n_stage - 1):
                wait_scatter(j, j % 4)

        @pl.when(c == 0)
        def _():
            for h in range(segs0):
                run_segment((s * segs0 + h) * n_stage)

        @pl.when(c == 1)
        def _():
            for h in range(segs1):
                run_segment((_NSUB * segs0 + s * segs1 + h) * n_stage)

        plsc.subcore_barrier()
        with jax.named_scope("readout"):
            pltpu.sync_copy(
                agg_s.at[pl.ds(s * rows_per_tile, rows_per_tile)],
                out_hbm.at[pl.ds(c * n_pad + s * rows_per_tile, rows_per_tile)])

    return k(y, src2d, dst2d, zeros)


def _matmul(x, w):
    n, kdim = x.shape
    m = w.shape[1]
    bn = 1000 if n % 1000 == 0 else n

    def body(x_ref, w_ref, o_ref):
        o_ref[...] = jnp.dot(x_ref[...], w_ref[...],
                             preferred_element_type=jnp.float32)

    return pl.pallas_call(
        body,
        grid=(n // bn,),
        in_specs=[
            pl.BlockSpec((bn, kdim), lambda i: (i, 0)),
            pl.BlockSpec((kdim, m), lambda i: (0, 0)),
        ],
        out_specs=pl.BlockSpec((bn, m), lambda i: (i, 0)),
        out_shape=jax.ShapeDtypeStruct((n, m), jnp.float32),
    )(x, w)


def _fuse_mm(y, parts, b, eps, w, n_pad):
    n, d = y.shape
    m = w.shape[1]
    bn = 1024
    nb = n_pad // bn

    def body(y_ref, p0_ref, p1_ref, b_ref, eps_ref, w_ref, o_ref):
        h = ((1.0 + eps_ref[0, 0]) * y_ref[...] + p0_ref[...] + p1_ref[...]
             + b_ref[...])
        h = jnp.maximum(h, 0.0)
        o_ref[...] = jnp.dot(h, w_ref[...], preferred_element_type=jnp.float32)

    return pl.pallas_call(
        body,
        grid=(pl.cdiv(n, bn),),
        in_specs=[
            pl.BlockSpec((bn, d), lambda i: (i, 0)),
            pl.BlockSpec((bn, d), lambda i: (i, 0)),
            pl.BlockSpec((bn, d), lambda i: (nb + i, 0)),
            pl.BlockSpec((1, d), lambda i: (0, 0)),
            pl.BlockSpec(memory_space=pltpu.SMEM),
            pl.BlockSpec((d, m), lambda i: (0, 0)),
        ],
        out_specs=pl.BlockSpec((bn, m), lambda i: (i, 0)),
        out_shape=jax.ShapeDtypeStruct((n, m), jnp.float32),
    )(y, parts, parts, b.reshape(1, d), eps.reshape(1, 1), w)


def _fuse_logsoftmax(y, parts, b, eps, n_pad):
    n, d = y.shape
    bn = 1024
    nb = n_pad // bn

    def body(y_ref, p0_ref, p1_ref, b_ref, eps_ref, o_ref):
        h = ((1.0 + eps_ref[0, 0]) * y_ref[...] + p0_ref[...] + p1_ref[...]
             + b_ref[...])
        mx = jnp.max(h, axis=1, keepdims=True)
        lse = jnp.log(jnp.sum(jnp.exp(h - mx), axis=1, keepdims=True)) + mx
        o_ref[...] = h - lse

    return pl.pallas_call(
        body,
        grid=(pl.cdiv(n, bn),),
        in_specs=[
            pl.BlockSpec((bn, d), lambda i: (i, 0)),
            pl.BlockSpec((bn, d), lambda i: (i, 0)),
            pl.BlockSpec((bn, d), lambda i: (nb + i, 0)),
            pl.BlockSpec((1, d), lambda i: (0, 0)),
            pl.BlockSpec(memory_space=pltpu.SMEM),
        ],
        out_specs=pl.BlockSpec((bn, d), lambda i: (i, 0)),
        out_shape=jax.ShapeDtypeStruct((n, d), jnp.float32),
    )(y, parts, parts, b.reshape(1, d), eps.reshape(1, 1))


def kernel(x, edge_index, W1, b1, eps1, W2, b2, eps2):
    n, d = x.shape
    e = edge_index.shape[1]
    h_dim = W1.shape[1]
    c_dim = W2.shape[1]

    n_pad = (n + 1 + 1024 - 1) // 1024 * 1024
    rows_per_tile = n_pad // _NSUB

    epb = _NW * _CHUNK * 80
    e_pad = (e + epb - 1) // epb * epb
    ei = edge_index.astype(jnp.int32)
    pad_i = jnp.arange(e_pad - e, dtype=jnp.int32)
    src = jnp.concatenate([ei[0], pad_i % n])
    dst = jnp.concatenate([ei[1], n + pad_i % (n_pad - n)])

    zeros_h = jnp.zeros((rows_per_tile, h_dim), jnp.float32)
    zeros_c = jnp.zeros((rows_per_tile, c_dim), jnp.float32)

    y1 = _matmul(x, W1)
    src2d = src.reshape(-1, 64)
    dst2d = dst.reshape(-1, 64)
    parts1 = _edge_agg(y1, src2d, dst2d, zeros_h, n_pad, 64, 2, 2)

    y2 = _fuse_mm(y1, parts1, b1, eps1, W2, n_pad)
    parts2 = _edge_agg(y2, src2d, dst2d, zeros_c, n_pad, 64, 2, 2)

    return _fuse_logsoftmax(y2, parts2, b2, eps2, n_pad)


# speedup vs baseline: 1.1101x; 1.1101x over previous
import functools

import jax
import jax.numpy as jnp
from jax import lax
from jax.experimental import pallas as pl
from jax.experimental.pallas import tpu as pltpu
from jax.experimental.pallas import tpu_sc as plsc

_CHUNK = 128
_NW = 32
_NSUB = 16


def _edge_agg(y, src2d, dst2d, zeros, n_pad, chunk, segs0, segs1, nbuf):
    n, d = y.shape
    t_chunks = src2d.shape[0]
    n_stage = t_chunks // (_NSUB * (segs0 + segs1))
    rows_per_tile = n_pad // _NSUB
    mesh = plsc.VectorSubcoreMesh(core_axis_name="c", subcore_axis_name="s")

    @functools.partial(
        pl.kernel,
        mesh=mesh,
        compiler_params=pltpu.CompilerParams(use_tc_tiling_on_sc=False),
        out_type=jax.ShapeDtypeStruct((2 * n_pad, d), jnp.float32),
        scratch_types=[
            pltpu.VMEM((n_stage, chunk), jnp.int32),
            pltpu.VMEM((n_stage, chunk), jnp.int32),
            pltpu.VMEM((nbuf, chunk, d), jnp.float32),
            pltpu.VMEM_SHARED((n_pad, d), jnp.float32),
            [pltpu.SemaphoreType.DMA] * nbuf,
            [pltpu.SemaphoreType.DMA] * nbuf,
        ],
    )
    def k(y_hbm, src_hbm, dst_hbm, z_hbm, out_hbm, src_v, dst_v, rows_v,
          agg_s, gsems, ssems):
        c = lax.axis_index("c")
        s = lax.axis_index("s")
        with jax.named_scope("zero"):
            pltpu.sync_copy(z_hbm,
                            agg_s.at[pl.ds(s * rows_per_tile, rows_per_tile)])
            plsc.subcore_barrier()

        def gather(j, p):
            pltpu.async_copy(y_hbm.at[src_v.at[j]], rows_v.at[p], gsems[p])

        def wait_gather(j, p):
            pltpu.make_async_copy(y_hbm.at[src_v.at[j]], rows_v.at[p],
                                  gsems[p]).wait()

        def scatter(j, p):
            pltpu.async_copy(rows_v.at[p], agg_s.at[dst_v.at[j]], ssems[p],
                             add=True)

        def wait_scatter(j, p):
            pltpu.make_async_copy(rows_v.at[p], agg_s.at[dst_v.at[j]],
                                  ssems[p]).wait()

        def run_segment(base):
            with jax.named_scope("stage"):
                pltpu.sync_copy(src_hbm.at[pl.ds(base, n_stage)], src_v)
                pltpu.sync_copy(dst_hbm.at[pl.ds(base, n_stage)], dst_v)

            ns = n_stage
            for j in range(nbuf - 2):
                gather(j, j)
            for j in range(nbuf - 2):
                wait_gather(j, j)
                scatter(j, j)
                if j >= 2:
                    wait_scatter(j - 2, (j - 2) % nbuf)
                gather(j + nbuf - 2, (j + nbuf - 2) % nbuf)

            steady = ns - 2 * nbuf + 4
            loops = steady // nbuf

            def step(j, p):
                wait_gather(j, p)
                scatter(j, p)
                wait_scatter(j - 2, (p + nbuf - 2) % nbuf)
                gather(j + nbuf - 2, (p + nbuf - 2) % nbuf)

            def body(jj, carry):
                for u in range(nbuf):
                    j = nbuf * jj + (nbuf - 2) + u
                    step(j, (nbuf - 2 + u) % nbuf)
                return carry

            with jax.named_scope("pipe"):
                lax.fori_loop(0, loops, body, 0)
            for j in range(nbuf - 2 + loops * nbuf, ns - nbuf + 2):
                step(j, j % nbuf)

            for j in range(ns - nbuf + 2, ns):
                p = j % nbuf
                wait_gather(j, p)
                scatter(j, p)
                wait_scatter(j - 2, (j - 2) % nbuf)
            for j in (ns - 2, ns - 1):
                wait_scatter(j, j % nbuf)

        @pl.when(c == 0)
        def _():
            for h in range(segs0):
                run_segment((s * segs0 + h) * n_stage)

        @pl.when(c == 1)
        def _():
            for h in range(segs1):
                run_segment((_NSUB * segs0 + s * segs1 + h) * n_stage)

        plsc.subcore_barrier()
        with jax.named_scope("readout"):
            pltpu.sync_copy(
                agg_s.at[pl.ds(s * rows_per_tile, rows_per_tile)],
                out_hbm.at[pl.ds(c * n_pad + s * rows_per_tile, rows_per_tile)])

    return k(y, src2d, dst2d, zeros)


def _matmul(x, w):
    n, kdim = x.shape
    m = w.shape[1]
    bn = 1000 if n % 1000 == 0 else n

    def body(x_ref, w_ref, o_ref):
        o_ref[...] = jnp.dot(x_ref[...], w_ref[...],
                             preferred_element_type=jnp.float32)

    return pl.pallas_call(
        body,
        grid=(n // bn,),
        in_specs=[
            pl.BlockSpec((bn, kdim), lambda i: (i, 0)),
            pl.BlockSpec((kdim, m), lambda i: (0, 0)),
        ],
        out_specs=pl.BlockSpec((bn, m), lambda i: (i, 0)),
        out_shape=jax.ShapeDtypeStruct((n, m), jnp.float32),
    )(x, w)


def _fuse_mm(y, parts, b, eps, w, n_pad):
    n, d = y.shape
    m = w.shape[1]
    bn = 1024
    nb = n_pad // bn

    def body(y_ref, p0_ref, p1_ref, b_ref, eps_ref, w_ref, o_ref):
        h = ((1.0 + eps_ref[0, 0]) * y_ref[...] + p0_ref[...] + p1_ref[...]
             + b_ref[...])
        h = jnp.maximum(h, 0.0)
        o_ref[...] = jnp.dot(h, w_ref[...], preferred_element_type=jnp.float32)

    return pl.pallas_call(
        body,
        grid=(pl.cdiv(n, bn),),
        in_specs=[
            pl.BlockSpec((bn, d), lambda i: (i, 0)),
            pl.BlockSpec((bn, d), lambda i: (i, 0)),
            pl.BlockSpec((bn, d), lambda i: (nb + i, 0)),
            pl.BlockSpec((1, d), lambda i: (0, 0)),
            pl.BlockSpec(memory_space=pltpu.SMEM),
            pl.BlockSpec((d, m), lambda i: (0, 0)),
        ],
        out_specs=pl.BlockSpec((bn, m), lambda i: (i, 0)),
        out_shape=jax.ShapeDtypeStruct((n, m), jnp.float32),
    )(y, parts, parts, b.reshape(1, d), eps.reshape(1, 1), w)


def _fuse_logsoftmax(y, parts, b, eps, n_pad):
    n, d = y.shape
    bn = 1024
    nb = n_pad // bn

    def body(y_ref, p0_ref, p1_ref, b_ref, eps_ref, o_ref):
        h = ((1.0 + eps_ref[0, 0]) * y_ref[...] + p0_ref[...] + p1_ref[...]
             + b_ref[...])
        mx = jnp.max(h, axis=1, keepdims=True)
        lse = jnp.log(jnp.sum(jnp.exp(h - mx), axis=1, keepdims=True)) + mx
        o_ref[...] = h - lse

    return pl.pallas_call(
        body,
        grid=(pl.cdiv(n, bn),),
        in_specs=[
            pl.BlockSpec((bn, d), lambda i: (i, 0)),
            pl.BlockSpec((bn, d), lambda i: (i, 0)),
            pl.BlockSpec((bn, d), lambda i: (nb + i, 0)),
            pl.BlockSpec((1, d), lambda i: (0, 0)),
            pl.BlockSpec(memory_space=pltpu.SMEM),
        ],
        out_specs=pl.BlockSpec((bn, d), lambda i: (i, 0)),
        out_shape=jax.ShapeDtypeStruct((n, d), jnp.float32),
    )(y, parts, parts, b.reshape(1, d), eps.reshape(1, 1))


def kernel(x, edge_index, W1, b1, eps1, W2, b2, eps2):
    n, d = x.shape
    e = edge_index.shape[1]
    h_dim = W1.shape[1]
    c_dim = W2.shape[1]

    n_pad = (n + 1 + 1024 - 1) // 1024 * 1024
    rows_per_tile = n_pad // _NSUB

    epb = _NW * _CHUNK * 80
    e_pad = (e + epb - 1) // epb * epb
    ei = edge_index.astype(jnp.int32)
    pad_i = jnp.arange(e_pad - e, dtype=jnp.int32)
    src = jnp.concatenate([ei[0], pad_i % n])
    dst = jnp.concatenate([ei[1], n + pad_i % (n_pad - n)])

    zeros_h = jnp.zeros((rows_per_tile, h_dim), jnp.float32)
    zeros_c = jnp.zeros((rows_per_tile, c_dim), jnp.float32)

    y1 = _matmul(x, W1)
    src2d = src.reshape(-1, 64)
    dst2d = dst.reshape(-1, 64)
    parts1 = _edge_agg(y1, src2d, dst2d, zeros_h, n_pad, 64, 4, 4, 5)

    y2 = _fuse_mm(y1, parts1, b1, eps1, W2, n_pad)
    parts2 = _edge_agg(y2, src.reshape(-1, _CHUNK), dst.reshape(-1, _CHUNK),
                       zeros_c, n_pad, _CHUNK, 2, 2, 4)

    return _fuse_logsoftmax(y2, parts2, b2, eps2, n_pad)


# speedup vs baseline: 1.1588x; 1.0438x over previous
import functools

import jax
import jax.numpy as jnp
from jax import lax
from jax.experimental import pallas as pl
from jax.experimental.pallas import tpu as pltpu
from jax.experimental.pallas import tpu_sc as plsc

_CHUNK = 128
_NW = 32
_NSUB = 16


def _edge_agg(y, src2d, dst2d, zeros, n_pad, chunk, segs0, segs1, nbuf):
    n, d = y.shape
    t_chunks = src2d.shape[0]
    n_stage = t_chunks // (_NSUB * (segs0 + segs1))
    rows_per_tile = n_pad // _NSUB
    mesh = plsc.VectorSubcoreMesh(core_axis_name="c", subcore_axis_name="s")

    @functools.partial(
        pl.kernel,
        mesh=mesh,
        compiler_params=pltpu.CompilerParams(use_tc_tiling_on_sc=False),
        out_type=jax.ShapeDtypeStruct((2 * n_pad, d), jnp.float32),
        scratch_types=[
            pltpu.VMEM((n_stage, chunk), jnp.int32),
            pltpu.VMEM((n_stage, chunk), jnp.int32),
            pltpu.VMEM((nbuf, chunk, d), jnp.float32),
            pltpu.VMEM_SHARED((n_pad, d), jnp.float32),
            [pltpu.SemaphoreType.DMA] * nbuf,
            [pltpu.SemaphoreType.DMA] * nbuf,
        ],
    )
    def k(y_hbm, src_hbm, dst_hbm, z_hbm, out_hbm, src_v, dst_v, rows_v,
          agg_s, gsems, ssems):
        c = lax.axis_index("c")
        s = lax.axis_index("s")
        with jax.named_scope("zero"):
            pltpu.sync_copy(z_hbm,
                            agg_s.at[pl.ds(s * rows_per_tile, rows_per_tile)])
            plsc.subcore_barrier()

        def gather(j, p):
            pltpu.async_copy(y_hbm.at[src_v.at[j]], rows_v.at[p], gsems[p])

        def wait_gather(j, p):
            pltpu.make_async_copy(y_hbm.at[src_v.at[j]], rows_v.at[p],
                                  gsems[p]).wait()

        def scatter(j, p):
            pltpu.async_copy(rows_v.at[p], agg_s.at[dst_v.at[j]], ssems[p],
                             add=True)

        def wait_scatter(j, p):
            pltpu.make_async_copy(rows_v.at[p], agg_s.at[dst_v.at[j]],
                                  ssems[p]).wait()

        def run_segment(base):
            with jax.named_scope("stage"):
                pltpu.sync_copy(src_hbm.at[pl.ds(base, n_stage)], src_v)
                pltpu.sync_copy(dst_hbm.at[pl.ds(base, n_stage)], dst_v)

            ns = n_stage
            for j in range(nbuf - 2):
                gather(j, j)
            for j in range(nbuf - 2):
                wait_gather(j, j)
                scatter(j, j)
                if j >= 2:
                    wait_scatter(j - 2, (j - 2) % nbuf)
                gather(j + nbuf - 2, (j + nbuf - 2) % nbuf)

            steady = ns - 2 * nbuf + 4
            loops = steady // nbuf

            def step(j, p):
                wait_gather(j, p)
                scatter(j, p)
                wait_scatter(j - 2, (p + nbuf - 2) % nbuf)
                gather(j + nbuf - 2, (p + nbuf - 2) % nbuf)

            def body(jj, carry):
                for u in range(nbuf):
                    j = nbuf * jj + (nbuf - 2) + u
                    step(j, (nbuf - 2 + u) % nbuf)
                return carry

            with jax.named_scope("pipe"):
                lax.fori_loop(0, loops, body, 0)
            for j in range(nbuf - 2 + loops * nbuf, ns - nbuf + 2):
                step(j, j % nbuf)

            for j in range(ns - nbuf + 2, ns):
                p = j % nbuf
                wait_gather(j, p)
                scatter(j, p)
                wait_scatter(j - 2, (j - 2) % nbuf)
            for j in (ns - 2, ns - 1):
                wait_scatter(j, j % nbuf)

        @pl.when(c == 0)
        def _():
            for h in range(segs0):
                run_segment((s * segs0 + h) * n_stage)

        @pl.when(c == 1)
        def _():
            for h in range(segs1):
                run_segment((_NSUB * segs0 + s * segs1 + h) * n_stage)

        plsc.subcore_barrier()
        with jax.named_scope("readout"):
            pltpu.sync_copy(
                agg_s.at[pl.ds(s * rows_per_tile, rows_per_tile)],
                out_hbm.at[pl.ds(c * n_pad + s * rows_per_tile, rows_per_tile)])

    return k(y, src2d, dst2d, zeros)


def _matmul(x, w):
    n, kdim = x.shape
    m = w.shape[1]
    bn = 1000 if n % 1000 == 0 else n

    def body(x_ref, w_ref, o_ref):
        o_ref[...] = jnp.dot(x_ref[...], w_ref[...],
                             preferred_element_type=jnp.float32)

    return pl.pallas_call(
        body,
        grid=(n // bn,),
        in_specs=[
            pl.BlockSpec((bn, kdim), lambda i: (i, 0)),
            pl.BlockSpec((kdim, m), lambda i: (0, 0)),
        ],
        out_specs=pl.BlockSpec((bn, m), lambda i: (i, 0)),
        out_shape=jax.ShapeDtypeStruct((n, m), jnp.float32),
    )(x, w)


def _fuse_mm(y, parts, b, eps, w, n_pad):
    n, d = y.shape
    m = w.shape[1]
    bn = 1024
    nb = n_pad // bn

    def body(y_ref, p0_ref, p1_ref, b_ref, eps_ref, w_ref, o_ref):
        h = ((1.0 + eps_ref[0, 0]) * y_ref[...] + p0_ref[...] + p1_ref[...]
             + b_ref[...])
        h = jnp.maximum(h, 0.0)
        o_ref[...] = jnp.dot(h, w_ref[...], preferred_element_type=jnp.float32)

    return pl.pallas_call(
        body,
        grid=(pl.cdiv(n, bn),),
        in_specs=[
            pl.BlockSpec((bn, d), lambda i: (i, 0)),
            pl.BlockSpec((bn, d), lambda i: (i, 0)),
            pl.BlockSpec((bn, d), lambda i: (nb + i, 0)),
            pl.BlockSpec((1, d), lambda i: (0, 0)),
            pl.BlockSpec(memory_space=pltpu.SMEM),
            pl.BlockSpec((d, m), lambda i: (0, 0)),
        ],
        out_specs=pl.BlockSpec((bn, m), lambda i: (i, 0)),
        out_shape=jax.ShapeDtypeStruct((n, m), jnp.float32),
    )(y, parts, parts, b.reshape(1, d), eps.reshape(1, 1), w)


def _fuse_logsoftmax(y, parts, b, eps, n_pad):
    n, d = y.shape
    bn = 1024
    nb = n_pad // bn

    def body(y_ref, p0_ref, p1_ref, b_ref, eps_ref, o_ref):
        h = ((1.0 + eps_ref[0, 0]) * y_ref[...] + p0_ref[...] + p1_ref[...]
             + b_ref[...])
        mx = jnp.max(h, axis=1, keepdims=True)
        lse = jnp.log(jnp.sum(jnp.exp(h - mx), axis=1, keepdims=True)) + mx
        o_ref[...] = h - lse

    return pl.pallas_call(
        body,
        grid=(pl.cdiv(n, bn),),
        in_specs=[
            pl.BlockSpec((bn, d), lambda i: (i, 0)),
            pl.BlockSpec((bn, d), lambda i: (i, 0)),
            pl.BlockSpec((bn, d), lambda i: (nb + i, 0)),
            pl.BlockSpec((1, d), lambda i: (0, 0)),
            pl.BlockSpec(memory_space=pltpu.SMEM),
        ],
        out_specs=pl.BlockSpec((bn, d), lambda i: (i, 0)),
        out_shape=jax.ShapeDtypeStruct((n, d), jnp.float32),
    )(y, parts, parts, b.reshape(1, d), eps.reshape(1, 1))


def kernel(x, edge_index, W1, b1, eps1, W2, b2, eps2):
    n, d = x.shape
    e = edge_index.shape[1]
    h_dim = W1.shape[1]
    c_dim = W2.shape[1]

    n_pad = (n + 1 + 1024 - 1) // 1024 * 1024
    rows_per_tile = n_pad // _NSUB

    epb = _NW * _CHUNK * 80
    e_pad = (e + epb - 1) // epb * epb
    ei = edge_index.astype(jnp.int32)
    pad_i = jnp.arange(e_pad - e, dtype=jnp.int32)
    src = jnp.concatenate([ei[0], pad_i % n])
    dst = jnp.concatenate([ei[1], n + pad_i % (n_pad - n)])

    zeros_h = jnp.zeros((rows_per_tile, h_dim), jnp.float32)
    zeros_c = jnp.zeros((rows_per_tile, c_dim), jnp.float32)

    y1 = _matmul(x, W1)
    src2d = src.reshape(-1, 64)
    dst2d = dst.reshape(-1, 64)
    parts1 = _edge_agg(y1, src2d, dst2d, zeros_h, n_pad, 64, 4, 4, 5)

    y2 = _fuse_mm(y1, parts1, b1, eps1, W2, n_pad)
    parts2 = _edge_agg(y2, src.reshape(-1, _CHUNK), dst.reshape(-1, _CHUNK),
                       zeros_c, n_pad, _CHUNK, 2, 2, 6)

    return _fuse_logsoftmax(y2, parts2, b2, eps2, n_pad)
